# baseline (speedup 1.0000x reference)
import functools

import jax
import jax.numpy as jnp
from jax import lax
from jax.experimental import pallas as pl
from jax.experimental.pallas import tpu as pltpu
from jax.experimental.pallas import tpu_sc as plsc

_EMB = 64
_SCALE = 8.0

_NC = 2
_NS = 16
_NW = _NC * _NS

_CHUNK = 512


@functools.lru_cache(maxsize=None)
def _emb_kernel(B, chunk):
    bpw = B // _NW
    nch = bpw // chunk
    mesh = plsc.VectorSubcoreMesh(core_axis_name="c", subcore_axis_name="s")

    @functools.partial(
        pl.kernel,
        mesh=mesh,
        compiler_params=pltpu.CompilerParams(use_tc_tiling_on_sc=False),
        out_type=jax.ShapeDtypeStruct((B, _EMB), jnp.float32),
        scratch_types=[
            pltpu.VMEM((chunk,), jnp.int32),
            pltpu.VMEM((chunk, _EMB), jnp.float32),
            pltpu.SemaphoreType.DMA,
        ],
    )
    def k(tok_hbm, table_hbm, out_hbm, idx_v, rows_v, gsem):
        wid = lax.axis_index("s") * _NC + lax.axis_index("c")
        base = wid * bpw

        def body(c, _):
            off = base + c * chunk
            pltpu.sync_copy(tok_hbm.at[pl.ds(off, chunk)], idx_v)
            pltpu.async_copy(table_hbm.at[idx_v], rows_v, gsem).wait()

            def srow(i, _):
                for j in range(_EMB // 16):
                    sl = pl.ds(j * 16, 16)
                    rows_v[i, sl] = rows_v[i, sl] * _SCALE
                return 0

            lax.fori_loop(0, chunk, srow, 0)
            pltpu.sync_copy(rows_v, out_hbm.at[pl.ds(off, chunk)])
            return 0

        lax.fori_loop(0, nch, body, 0)

    return k


@jax.jit
def kernel(tokens, table):
    r, c = tokens.shape
    b = r * c
    tok = tokens.reshape(b).astype(jnp.int32)
    out = _emb_kernel(b, _CHUNK)(tok, table)
    return out.reshape(r, c, _EMB)


# speedup vs baseline: 1.2210x; 1.2210x over previous
import functools

import jax
import jax.numpy as jnp
from jax import lax
from jax.experimental import pallas as pl
from jax.experimental.pallas import tpu as pltpu
from jax.experimental.pallas import tpu_sc as plsc

_EMB = 64
_SCALE = 8.0

_NC = 2
_NS = 16
_NW = _NC * _NS

_CHUNK = 400
_NBUF = 4


@functools.lru_cache(maxsize=None)
def _emb_kernel(B, chunk, nbuf):
    bpw = B // _NW
    nch = bpw // chunk
    assert nch % nbuf == 0 and nch >= nbuf
    mesh = plsc.VectorSubcoreMesh(core_axis_name="c", subcore_axis_name="s")

    @functools.partial(
        pl.kernel,
        mesh=mesh,
        compiler_params=pltpu.CompilerParams(use_tc_tiling_on_sc=False),
        out_type=jax.ShapeDtypeStruct((B, _EMB), jnp.float32),
        scratch_types=[
            pltpu.VMEM((nbuf, chunk), jnp.int32),
            pltpu.VMEM((nbuf, chunk, _EMB), jnp.float32),
            pltpu.SemaphoreType.DMA((nbuf,)),
            pltpu.SemaphoreType.DMA((nbuf,)),
        ],
    )
    def k(tok_hbm, table_hbm, out_hbm, idx_v, rows_v, gsem, ssem):
        wid = lax.axis_index("s") * _NC + lax.axis_index("c")
        base = wid * bpw

        def start_gather(ck, b):
            off = base + ck * chunk
            pltpu.sync_copy(tok_hbm.at[pl.ds(off, chunk)], idx_v.at[b])
            pltpu.async_copy(table_hbm.at[idx_v.at[b]], rows_v.at[b],
                             gsem.at[b])

        for b in range(2):
            start_gather(b, b)

        def body(g, _):
            for b in range(nbuf):
                ck = g * nbuf + b

                b2 = (b + 2) % nbuf

                @pl.when(ck >= 2)
                def _():
                    off2 = base + (ck - 2) * chunk
                    pltpu.make_async_copy(
                        rows_v.at[b2], out_hbm.at[pl.ds(off2, chunk)],
                        ssem.at[b2]).wait()

                @pl.when(ck < nch - 2)
                def _():
                    start_gather(ck + 2, b2)

                off = base + ck * chunk
                pltpu.make_async_copy(
                    table_hbm.at[idx_v.at[b]], rows_v.at[b],
                    gsem.at[b]).wait()

                @plsc.parallel_loop(0, chunk, 1, unroll=8)
                def _(i):
                    for j in range(_EMB // 16):
                        sl = pl.ds(j * 16, 16)
                        rows_v[b, i, sl] = rows_v[b, i, sl] * _SCALE

                pltpu.async_copy(rows_v.at[b], out_hbm.at[pl.ds(off, chunk)],
                                 ssem.at[b])
            return 0

        lax.fori_loop(0, nch // nbuf, body, 0)

        for ck in (nch - 2, nch - 1):
            b = ck % nbuf
            off = base + ck * chunk
            pltpu.make_async_copy(
                rows_v.at[b], out_hbm.at[pl.ds(off, chunk)],
                ssem.at[b]).wait()

    return k


@jax.jit
def kernel(tokens, table):
    r, c = tokens.shape
    b = r * c
    tok = tokens.reshape(b).astype(jnp.int32)
    out = _emb_kernel(b, _CHUNK, _NBUF)(tok, table)
    return out.reshape(r, c, _EMB)
